# baseline (speedup 1.0000x reference)
import functools

import jax
import jax.numpy as jnp
from jax import lax
from jax.experimental import pallas as pl
from jax.experimental.pallas import tpu as pltpu
from jax.experimental.pallas import tpu_sc as plsc

B = 4096
L = 200
EMB = 32
N = B * L

_info = plsc.get_sparse_core_info()
NC = _info.num_cores
NS = _info.num_subcores
NW = NC * NS
BT = B // 128
NUNIT = L * BT
PER_W = NUNIT // NW
PER_W_IDX = PER_W * 128
NSLOT = 4

_mesh = plsc.VectorSubcoreMesh(core_axis_name="c", subcore_axis_name="s")

NITEM = 1000000
NGRP = 999808 // 128
NITEM_PAD = (NGRP + 2) * 128
GPW = NGRP // (NW // 2)
GREM = NGRP - GPW * (NW // 2)


@functools.partial(
    pl.kernel,
    mesh=_mesh,
    compiler_params=pltpu.CompilerParams(
        use_tc_tiling_on_sc=True, needs_layout_passes=False),
    out_type=(jax.ShapeDtypeStruct((NITEM_PAD // 4, 128), jnp.float32),
              jax.ShapeDtypeStruct((NITEM_PAD // 4, 128), jnp.float32)),
    scratch_types=[
        pltpu.VMEM((EMB, 128), jnp.float32),
        pltpu.VMEM((EMB, 128), jnp.float32),
    ],
)
def _sc_detile(t0_hbm, t1_hbm, t0t_hbm, t1t_hbm, o0_hbm, o1_hbm, ibuf, obuf):
    wid = lax.axis_index("s") * NC + lax.axis_index("c")
    wl = wid & 15
    base = wl * GPW + jnp.minimum(wl, GREM)
    ng = GPW + (wl < GREM).astype(jnp.int32)

    iota = lax.iota(jnp.int32, 16)
    diag = [(iota + d) & 15 for d in range(16)]

    def load_transpose(src_ref, lane0):
        for ft in range(4):
            pltpu.sync_copy(src_ref.at[pl.ds(ft * 8, 8), pl.ds(lane0, 128)],
                            ibuf.at[pl.ds(ft * 8, 8)])

        def isub(s, carry2):
            items = s * 16 + iota
            for f0 in (0, 16):
                for d0 in range(0, 16, 4):
                    fv = [diag[d0 + k] + f0 for k in range(4)]
                    vs = [plsc.load_gather(ibuf, [f, items]) for f in fv]
                    for f, v in zip(fv, vs):
                        dst = items * EMB + f
                        plsc.store_scatter(obuf, [dst >> 7, dst & 127], v)
            return carry2

        lax.fori_loop(0, 8, isub, 0)

    def do_groups(t_hbm, tt_hbm, o_hbm):
        def grp(g, carry):
            gi = base + g
            load_transpose(t_hbm, gi * 128)
            pltpu.sync_copy(obuf, o_hbm.at[pl.ds(gi * 32, 32)])
            return carry

        lax.fori_loop(0, ng, grp, 0)

        @pl.when(wl == 15)
        def _():
            for tg in range(2):
                load_transpose(tt_hbm, tg * 128)
                pltpu.sync_copy(obuf, o_hbm.at[pl.ds((NGRP + tg) * 32, 32)])

    @pl.when(wid < 16)
    def _():
        do_groups(t0_hbm, t0t_hbm, o0_hbm)

    @pl.when(wid >= 16)
    def _():
        do_groups(t1_hbm, t1t_hbm, o1_hbm)


@functools.partial(
    pl.kernel,
    mesh=_mesh,
    compiler_params=pltpu.CompilerParams(
        use_tc_tiling_on_sc=False, needs_layout_passes=False),
    out_type=jax.ShapeDtypeStruct((L, 8, BT, 8, 128), jnp.float32),
    scratch_types=[
        pltpu.VMEM((PER_W_IDX,), jnp.int32),
        pltpu.VMEM((PER_W_IDX,), jnp.int32),
        [pltpu.VMEM((128, EMB), jnp.float32) for _ in range(NSLOT)],
        [pltpu.VMEM((128, EMB), jnp.float32) for _ in range(NSLOT)],
        [pltpu.VMEM((2 * EMB, 128), jnp.float32) for _ in range(NSLOT)],
        [pltpu.SemaphoreType.DMA for _ in range(NSLOT)],
        [pltpu.SemaphoreType.DMA for _ in range(NSLOT)],
    ],
)
def _sc_gather(idx0_hbm, idx1_hbm, t0_hbm, t1_hbm, out_hbm,
               idx0_v, idx1_v, g0, g1, nb, gsem, wsem):
    wid = lax.axis_index("s") * NC + lax.axis_index("c")
    ubase = wid * PER_W

    pltpu.sync_copy(idx0_hbm.at[pl.ds(ubase * 128, PER_W_IDX)], idx0_v)
    pltpu.sync_copy(idx1_hbm.at[pl.ds(ubase * 128, PER_W_IDX)], idx1_v)

    iota = lax.iota(jnp.int32, 16)
    diag = [(iota + d) & 15 for d in range(16)]

    def fire_gathers(u, p):
        pltpu.async_copy(t0_hbm.at[idx0_v.at[pl.ds(u * 128, 128)]], g0[p], gsem[p])
        pltpu.async_copy(t1_hbm.at[idx1_v.at[pl.ds(u * 128, 128)]], g1[p], gsem[p])

    def wait_gathers(p):
        pltpu.make_async_copy(t0_hbm.at[pl.ds(0, 128)], g0[p], gsem[p]).wait()
        pltpu.make_async_copy(t1_hbm.at[pl.ds(0, 128)], g1[p], gsem[p]).wait()

    def fire_write(u, p):
        gu = ubase + u
        l = gu // BT
        bt = gu % BT
        for et in range(8):
            pltpu.async_copy(nb[p].at[pl.ds(et * 8, 8)], out_hbm.at[l, et, bt],
                             wsem[p])

    def wait_write(p):
        for et in range(8):
            pltpu.make_async_copy(nb[p].at[pl.ds(0, 8)], out_hbm.at[0, 0, 0],
                                  wsem[p]).wait()

    def assemble(p):
        def sub(rb, carry):
            items = rb * 16 + iota
            for g, ebase in ((g0[p], 0), (g1[p], EMB)):
                for f0 in range(0, EMB, 16):
                    for d0 in range(0, 16, 4):
                        fv = [diag[d0 + k] + f0 for k in range(4)]
                        vs = [plsc.load_gather(g, [items, f]) for f in fv]
                        for f, v in zip(fv, vs):
                            plsc.store_scatter(nb[p], [f + ebase, items], v)
            return carry

        lax.fori_loop(0, 8, sub, 0)

    for p in range(NSLOT):
        fire_gathers(p, p)

    for i in range(NSLOT):
        wait_gathers(i)
        assemble(i)
        fire_write(i, i)
        fire_gathers(i + NSLOT, i)

    def round_body(r, carry):
        for i in range(NSLOT):
            u = r * NSLOT + i
            wait_gathers(i)
            wait_write(i)
            assemble(i)
            fire_write(u, i)
            fire_gathers(u + NSLOT, i)
        return carry

    lax.fori_loop(1, PER_W // NSLOT - 1, round_body, 0)

    for i in range(NSLOT):
        u = PER_W - NSLOT + i
        wait_gathers(i)
        wait_write(i)
        assemble(i)
        fire_write(u, i)
    for i in range(NSLOT):
        wait_write(i)


def _mask_body(h_ref, m_ref):
    m_ref[...] = h_ref[...] != 0


_mask_call = pl.pallas_call(
    _mask_body,
    out_shape=jax.ShapeDtypeStruct((B, L), jnp.bool_),
)


def kernel(histories, item_emb_0, item_emb_1):
    hist0 = histories[:, 0, :]
    idx0 = jnp.swapaxes(hist0, 0, 1).reshape(N)
    idx1 = jnp.swapaxes(histories[:, 1, :], 0, 1).reshape(N)
    t0tail = jnp.pad(item_emb_0[NGRP * 128:], ((0, 64), (0, 0))).T
    t1tail = jnp.pad(item_emb_1[NGRP * 128:], ((0, 64), (0, 0))).T
    t0lin, t1lin = _sc_detile(item_emb_0.T, item_emb_1.T, t0tail, t1tail)
    out5 = _sc_gather(idx0, idx1,
                      t0lin.reshape(NITEM_PAD, EMB),
                      t1lin.reshape(NITEM_PAD, EMB))
    embs = out5.transpose(2, 4, 0, 1, 3).reshape(B, L, 2 * EMB)
    mask = _mask_call(hist0)
    return embs, mask


# speedup vs baseline: 2.6796x; 2.6796x over previous
import functools

import jax
import jax.numpy as jnp
from jax import lax
from jax.experimental import pallas as pl
from jax.experimental.pallas import tpu as pltpu
from jax.experimental.pallas import tpu_sc as plsc

B = 4096
L = 200
EMB = 32
N = B * L

_info = plsc.get_sparse_core_info()
NC = _info.num_cores
NS = _info.num_subcores
NW = NC * NS
BT = B // 128
NUNIT = L * BT
PER_W = NUNIT // NW
PER_W_IDX = PER_W * 128
NSLOT = 4

_mesh = plsc.VectorSubcoreMesh(core_axis_name="c", subcore_axis_name="s")

NITEM = 1000000
NGRP = 999808 // 128
NITEM_PAD = (NGRP + 2) * 128
GPW = NGRP // (NW // 2)
GREM = NGRP - GPW * (NW // 2)


@functools.partial(
    pl.kernel,
    mesh=_mesh,
    compiler_params=pltpu.CompilerParams(
        use_tc_tiling_on_sc=True, needs_layout_passes=False),
    out_type=(jax.ShapeDtypeStruct((NITEM_PAD // 4, 128), jnp.float32),
              jax.ShapeDtypeStruct((NITEM_PAD // 4, 128), jnp.float32)),
    scratch_types=[
        [pltpu.VMEM((EMB, 128), jnp.float32) for _ in range(2)],
        [pltpu.VMEM((EMB, 128), jnp.float32) for _ in range(2)],
        [pltpu.SemaphoreType.DMA for _ in range(2)],
        [pltpu.SemaphoreType.DMA for _ in range(2)],
    ],
)
def _sc_detile(t0_hbm, t1_hbm, t0t_hbm, t1t_hbm, o0_hbm, o1_hbm,
               ibuf, obuf, isem, osem):
    wid = lax.axis_index("s") * NC + lax.axis_index("c")
    wl = wid & 15
    base = wl * GPW

    iota = lax.iota(jnp.int32, 16)
    diag = [(iota + d) & 15 for d in range(16)]

    def fire_in(t_hbm, lane0, p):
        for ft in range(4):
            pltpu.async_copy(t_hbm.at[pl.ds(ft * 8, 8), pl.ds(lane0, 128)],
                             ibuf[p].at[pl.ds(ft * 8, 8)], isem[p])

    def wait_in(t_hbm, p):
        for ft in range(4):
            pltpu.make_async_copy(t_hbm.at[pl.ds(0, 8), pl.ds(0, 128)],
                                  ibuf[p].at[pl.ds(0, 8)], isem[p]).wait()

    def fire_out(o_hbm, gi, p):
        pltpu.async_copy(obuf[p], o_hbm.at[pl.ds(gi * 32, 32)], osem[p])

    def wait_out(o_hbm, p):
        pltpu.make_async_copy(obuf[p], o_hbm.at[pl.ds(0, 32)], osem[p]).wait()

    def transpose(p):
        def isub(s, carry2):
            items = s * 16 + iota
            for f0 in (0, 16):
                for d0 in range(0, 16, 4):
                    fv = [diag[d0 + k] + f0 for k in range(4)]
                    vs = [plsc.load_gather(ibuf[p], [f, items]) for f in fv]
                    for f, v in zip(fv, vs):
                        dst = items * EMB + f
                        plsc.store_scatter(obuf[p], [dst >> 7, dst & 127], v)
            return carry2

        lax.fori_loop(0, 8, isub, 0)

    def do_groups(t_hbm, tt_hbm, o_hbm):
        def step(u, p, first, refill):
            wait_in(t_hbm, p)
            if not first:
                wait_out(o_hbm, p)
            transpose(p)
            fire_out(o_hbm, base + u, p)
            if refill:
                fire_in(t_hbm, (base + u + 2) * 128, p)

        fire_in(t_hbm, base * 128, 0)
        fire_in(t_hbm, (base + 1) * 128, 1)
        step(0, 0, True, True)
        step(1, 1, True, True)

        def rnd(r, carry):
            step(2 * r, 0, False, True)
            step(2 * r + 1, 1, False, True)
            return carry

        lax.fori_loop(1, GPW // 2 - 1, rnd, 0)
        for u in (GPW - 2, GPW - 1):
            p = u & 1
            wait_in(t_hbm, p)
            wait_out(o_hbm, p)
            transpose(p)
            fire_out(o_hbm, base + u, p)

        @pl.when(wl < GREM)
        def _():
            gi = NGRP - GREM + wl
            wait_out(o_hbm, 0)
            fire_in(t_hbm, gi * 128, 0)
            wait_in(t_hbm, 0)
            transpose(0)
            fire_out(o_hbm, gi, 0)

        @pl.when(wl == 15)
        def _():
            for tg in range(2):
                wait_out(o_hbm, 0)
                fire_in(tt_hbm, tg * 128, 0)
                wait_in(tt_hbm, 0)
                transpose(0)
                fire_out(o_hbm, NGRP + tg, 0)

        wait_out(o_hbm, 0)
        wait_out(o_hbm, 1)

    @pl.when(wid < 16)
    def _():
        do_groups(t0_hbm, t0t_hbm, o0_hbm)

    @pl.when(wid >= 16)
    def _():
        do_groups(t1_hbm, t1t_hbm, o1_hbm)


@functools.partial(
    pl.kernel,
    mesh=_mesh,
    compiler_params=pltpu.CompilerParams(
        use_tc_tiling_on_sc=False, needs_layout_passes=False),
    out_type=jax.ShapeDtypeStruct((L, 8, BT, 8, 128), jnp.float32),
    scratch_types=[
        pltpu.VMEM((PER_W_IDX,), jnp.int32),
        pltpu.VMEM((PER_W_IDX,), jnp.int32),
        [pltpu.VMEM((128, EMB), jnp.float32) for _ in range(NSLOT)],
        [pltpu.VMEM((128, EMB), jnp.float32) for _ in range(NSLOT)],
        [pltpu.VMEM((2 * EMB, 128), jnp.float32) for _ in range(NSLOT)],
        [pltpu.SemaphoreType.DMA for _ in range(NSLOT)],
        [pltpu.SemaphoreType.DMA for _ in range(NSLOT)],
    ],
)
def _sc_gather(idx0_hbm, idx1_hbm, t0_hbm, t1_hbm, out_hbm,
               idx0_v, idx1_v, g0, g1, nb, gsem, wsem):
    wid = lax.axis_index("s") * NC + lax.axis_index("c")
    ubase = wid * PER_W

    pltpu.sync_copy(idx0_hbm.at[pl.ds(ubase * 128, PER_W_IDX)], idx0_v)
    pltpu.sync_copy(idx1_hbm.at[pl.ds(ubase * 128, PER_W_IDX)], idx1_v)

    iota = lax.iota(jnp.int32, 16)
    diag = [(iota + d) & 15 for d in range(16)]

    def fire_gathers(u, p):
        pltpu.async_copy(t0_hbm.at[idx0_v.at[pl.ds(u * 128, 128)]], g0[p], gsem[p])
        pltpu.async_copy(t1_hbm.at[idx1_v.at[pl.ds(u * 128, 128)]], g1[p], gsem[p])

    def wait_gathers(p):
        pltpu.make_async_copy(t0_hbm.at[pl.ds(0, 128)], g0[p], gsem[p]).wait()
        pltpu.make_async_copy(t1_hbm.at[pl.ds(0, 128)], g1[p], gsem[p]).wait()

    def fire_write(u, p):
        gu = ubase + u
        l = gu // BT
        bt = gu % BT
        for et in range(8):
            pltpu.async_copy(nb[p].at[pl.ds(et * 8, 8)], out_hbm.at[l, et, bt],
                             wsem[p])

    def wait_write(p):
        for et in range(8):
            pltpu.make_async_copy(nb[p].at[pl.ds(0, 8)], out_hbm.at[0, 0, 0],
                                  wsem[p]).wait()

    def assemble(p):
        def sub(rb, carry):
            items = rb * 16 + iota
            for g, ebase in ((g0[p], 0), (g1[p], EMB)):
                for f0 in range(0, EMB, 16):
                    for d0 in range(0, 16, 4):
                        fv = [diag[d0 + k] + f0 for k in range(4)]
                        vs = [plsc.load_gather(g, [items, f]) for f in fv]
                        for f, v in zip(fv, vs):
                            plsc.store_scatter(nb[p], [f + ebase, items], v)
            return carry

        lax.fori_loop(0, 8, sub, 0)

    for p in range(NSLOT):
        fire_gathers(p, p)

    for i in range(NSLOT):
        wait_gathers(i)
        assemble(i)
        fire_write(i, i)
        fire_gathers(i + NSLOT, i)

    def round_body(r, carry):
        for i in range(NSLOT):
            u = r * NSLOT + i
            wait_gathers(i)
            wait_write(i)
            assemble(i)
            fire_write(u, i)
            fire_gathers(u + NSLOT, i)
        return carry

    lax.fori_loop(1, PER_W // NSLOT - 1, round_body, 0)

    for i in range(NSLOT):
        u = PER_W - NSLOT + i
        wait_gathers(i)
        wait_write(i)
        assemble(i)
        fire_write(u, i)
    for i in range(NSLOT):
        wait_write(i)


def _mask_body(h_ref, m_ref):
    m_ref[...] = h_ref[...] != 0


_mask_call = pl.pallas_call(
    _mask_body,
    out_shape=jax.ShapeDtypeStruct((B, L), jnp.bool_),
)


def kernel(histories, item_emb_0, item_emb_1):
    hist0 = histories[:, 0, :]
    idx0 = jnp.swapaxes(hist0, 0, 1).reshape(N)
    idx1 = jnp.swapaxes(histories[:, 1, :], 0, 1).reshape(N)
    t0tail = jnp.pad(item_emb_0[NGRP * 128:], ((0, 64), (0, 0))).T
    t1tail = jnp.pad(item_emb_1[NGRP * 128:], ((0, 64), (0, 0))).T
    t0lin, t1lin = _sc_detile(item_emb_0.T, item_emb_1.T, t0tail, t1tail)
    out5 = _sc_gather(idx0, idx1,
                      t0lin.reshape(NITEM_PAD, EMB),
                      t1lin.reshape(NITEM_PAD, EMB))
    embs = out5.transpose(2, 4, 0, 1, 3).reshape(B, L, 2 * EMB)
    mask = _mask_call(hist0)
    return embs, mask
